# baseline (device time: 187690 ns/iter reference)
import jax
import jax.numpy as jnp
from jax import lax
from jax.experimental import pallas as pl
from jax.experimental.pallas import tpu as pltpu

N_Y = 4
H = 16
D = 64
HD = H * D
B = 16
NBUF = 6


def _expansion_mask(dtype):
    r = lax.broadcasted_iota(jnp.int32, (H, HD), 0)
    c = lax.broadcasted_iota(jnp.int32, (H, HD), 1)
    return (c // D == r).astype(dtype)


def _body(q_ref, k_hbm, v_hbm, out_ref,
          k_buf, v_buf, m_c, l_c, o_c,
          k_sems, v_sems, send_sems, recv_sems):
    my_x = lax.axis_index("x")
    my_y = lax.axis_index("y")
    my_z = lax.axis_index("z")

    e_bf = _expansion_mask(jnp.bfloat16)
    e_f32 = _expansion_mask(jnp.float32)
    scale = D ** -0.5

    def copy_for(b):
        slot = b % NBUF
        kcp = pltpu.make_async_copy(k_hbm.at[b], k_buf.at[slot],
                                    k_sems.at[slot])
        vcp = pltpu.make_async_copy(v_hbm.at[b], v_buf.at[slot],
                                    v_sems.at[slot])
        return kcp, vcp

    for b in range(NBUF):
        kcp, vcp = copy_for(b)
        kcp.start()
        vcp.start()

    for b in range(B):
        slot = b % NBUF
        kcp, vcp = copy_for(b)
        kcp.wait()
        vcp.wait()

        k = k_buf[slot].astype(jnp.bfloat16)
        v = v_buf[slot].astype(jnp.bfloat16)
        q = q_ref[b].astype(jnp.bfloat16)

        a = e_bf * q
        s = lax.dot_general(
            k, a,
            dimension_numbers=(((1,), (1,)), ((), ())),
            preferred_element_type=jnp.float32,
        ) * scale
        m = jnp.max(s, axis=0, keepdims=True)
        p = jnp.exp(s - m)
        l = jnp.sum(p, axis=0, keepdims=True)
        o_full = lax.dot_general(
            p.astype(jnp.bfloat16), v,
            dimension_numbers=(((0,), (0,)), ((), ())),
            preferred_element_type=jnp.float32,
        )
        o = jnp.sum(o_full * e_f32, axis=0, keepdims=True)

        if b + NBUF < B:
            kcp2, vcp2 = copy_for(b + NBUF)
            kcp2.start()
            vcp2.start()

        m_c[my_y, b] = m[0]
        l_c[my_y, b] = l[0]
        o_c[my_y, b] = o[0]

    barrier = pltpu.get_barrier_semaphore()
    for dy in range(1, N_Y):
        peer_y = (my_y + dy) % N_Y
        pl.semaphore_signal(
            barrier, inc=1,
            device_id=(my_x, peer_y, my_z),
            device_id_type=pl.DeviceIdType.MESH,
        )
    pl.semaphore_wait(barrier, N_Y - 1)

    bufs = (m_c, l_c, o_c)
    sends = []
    for dy in range(1, N_Y):
        peer_y = (my_y + dy) % N_Y
        for ti, buf in enumerate(bufs):
            rdma = pltpu.make_async_remote_copy(
                src_ref=buf.at[my_y],
                dst_ref=buf.at[my_y],
                send_sem=send_sems.at[dy - 1, ti],
                recv_sem=recv_sems.at[my_y, ti],
                device_id=(my_x, peer_y, my_z),
                device_id_type=pl.DeviceIdType.MESH,
            )
            rdma.start()
            sends.append(rdma)
    for rdma in sends:
        rdma.wait_send()

    for dy in range(1, N_Y):
        src_y = (my_y + dy) % N_Y
        for ti, buf in enumerate(bufs):
            recv = pltpu.make_async_remote_copy(
                src_ref=buf.at[src_y],
                dst_ref=buf.at[src_y],
                send_sem=send_sems.at[dy - 1, ti],
                recv_sem=recv_sems.at[src_y, ti],
                device_id=(my_x, my_y, my_z),
                device_id_type=pl.DeviceIdType.MESH,
            )
            recv.wait_recv()

    m_all = m_c[...]
    mx = jnp.max(m_all, axis=0)
    w = jnp.exp(m_all - mx[None])
    l_tot = jnp.sum(l_c[...] * w, axis=0)
    w_hd = jnp.reshape(
        lax.dot_general(
            jnp.reshape(w, (N_Y * B, H)), e_f32,
            dimension_numbers=(((1,), (0,)), ((), ())),
            preferred_element_type=jnp.float32,
        ),
        (N_Y, B, HD),
    )
    o_tot = jnp.sum(o_c[...] * w_hd, axis=0)
    l_hd = lax.dot_general(
        l_tot, e_f32,
        dimension_numbers=(((1,), (0,)), ((), ())),
        preferred_element_type=jnp.float32,
    )
    out_ref[...] = o_tot / l_hd


def kernel(Q, K, V):
    b, skv, h, d = K.shape
    hd = h * d
    Q2 = jnp.reshape(Q, (b, 1, hd))
    K2 = jnp.reshape(K, (b, skv, hd))
    V2 = jnp.reshape(V, (b, skv, hd))
    out = pl.pallas_call(
        _body,
        in_specs=[
            pl.BlockSpec(memory_space=pltpu.VMEM),
            pl.BlockSpec(memory_space=pl.ANY),
            pl.BlockSpec(memory_space=pl.ANY),
        ],
        out_specs=pl.BlockSpec(memory_space=pltpu.VMEM),
        out_shape=jax.ShapeDtypeStruct((b, hd), jnp.float32),
        scratch_shapes=[
            pltpu.VMEM((NBUF, skv, hd), jnp.float32),
            pltpu.VMEM((NBUF, skv, hd), jnp.float32),
            pltpu.VMEM((N_Y, b, H), jnp.float32),
            pltpu.VMEM((N_Y, b, H), jnp.float32),
            pltpu.VMEM((N_Y, b, hd), jnp.float32),
            pltpu.SemaphoreType.DMA((NBUF,)),
            pltpu.SemaphoreType.DMA((NBUF,)),
            pltpu.SemaphoreType.DMA((N_Y - 1, 3)),
            pltpu.SemaphoreType.DMA((N_Y, 3)),
        ],
        compiler_params=pltpu.CompilerParams(
            collective_id=0,
            vmem_limit_bytes=110 * 1024 * 1024,
        ),
    )(Q2, K2, V2)
    return jnp.reshape(out, (b, 1, h, d))


# device time: 155176 ns/iter; 1.2095x vs baseline; 1.2095x over previous
import jax
import jax.numpy as jnp
from jax import lax
from jax.experimental import pallas as pl
from jax.experimental.pallas import tpu as pltpu

N_X = 2
N_Y = 4
N_Z = 4
N_XZ = N_X * N_Z
H = 16
D = 64
HD = H * D
B = 16
BL = B // N_XZ


def _expansion_mask(dtype):
    r = lax.broadcasted_iota(jnp.int32, (H, HD), 0)
    c = lax.broadcasted_iota(jnp.int32, (H, HD), 1)
    return (c // D == r).astype(dtype)


def _body(q_ref, k_hbm, v_hbm, out_ref,
          k_buf, v_buf, m_c, l_c, o_c,
          k_sems, v_sems, y_send_sems, y_recv_sems,
          xz_send_sems, xz_recv_sems):
    my_x = lax.axis_index("x")
    my_y = lax.axis_index("y")
    my_z = lax.axis_index("z")
    my_r = my_x * N_Z + my_z
    b0 = my_r * BL

    e_bf = _expansion_mask(jnp.bfloat16)
    e_f32 = _expansion_mask(jnp.float32)
    scale = D ** -0.5

    copies = []
    for i in range(BL):
        kcp = pltpu.make_async_copy(k_hbm.at[b0 + i], k_buf.at[i],
                                    k_sems.at[i])
        vcp = pltpu.make_async_copy(v_hbm.at[b0 + i], v_buf.at[i],
                                    v_sems.at[i])
        kcp.start()
        vcp.start()
        copies.append((kcp, vcp))

    parts = []
    for i in range(BL):
        kcp, vcp = copies[i]
        kcp.wait()
        vcp.wait()
        k = k_buf[i].astype(jnp.bfloat16)
        v = v_buf[i].astype(jnp.bfloat16)
        q = q_ref[pl.ds(b0 + i, 1), 0, :].astype(jnp.bfloat16)

        a = e_bf * q
        s = lax.dot_general(
            k, a,
            dimension_numbers=(((1,), (1,)), ((), ())),
            preferred_element_type=jnp.float32,
        ) * scale
        m = jnp.max(s, axis=0, keepdims=True)
        p = jnp.exp(s - m)
        l = jnp.sum(p, axis=0, keepdims=True)
        o_full = lax.dot_general(
            p.astype(jnp.bfloat16), v,
            dimension_numbers=(((0,), (0,)), ((), ())),
            preferred_element_type=jnp.float32,
        )
        o = jnp.sum(o_full * e_f32, axis=0, keepdims=True)
        parts.append((m, l, o))

    m_c[my_y] = jnp.concatenate([p[0] for p in parts], axis=0)
    l_c[my_y] = jnp.concatenate([p[1] for p in parts], axis=0)
    o_c[my_y] = jnp.concatenate([p[2] for p in parts], axis=0)

    y_peers = [((my_y + dy) % N_Y) for dy in range(1, N_Y)]
    xz_peers = []
    for dx in range(N_X):
        for dz in range(N_Z):
            if dx == 0 and dz == 0:
                continue
            xz_peers.append(((my_x + dx) % N_X, (my_z + dz) % N_Z))

    barrier = pltpu.get_barrier_semaphore()
    for py in y_peers:
        pl.semaphore_signal(barrier, inc=1, device_id=(my_x, py, my_z),
                            device_id_type=pl.DeviceIdType.MESH)
    for px, pz in xz_peers:
        pl.semaphore_signal(barrier, inc=1, device_id=(px, my_y, pz),
                            device_id_type=pl.DeviceIdType.MESH)
    pl.semaphore_wait(barrier, N_Y - 1 + N_XZ - 1)

    bufs = (m_c, l_c, o_c)
    sends = []
    for di, py in enumerate(y_peers):
        for ti, buf in enumerate(bufs):
            rdma = pltpu.make_async_remote_copy(
                src_ref=buf.at[my_y],
                dst_ref=buf.at[my_y],
                send_sem=y_send_sems.at[di, ti],
                recv_sem=y_recv_sems.at[my_y, ti],
                device_id=(my_x, py, my_z),
                device_id_type=pl.DeviceIdType.MESH,
            )
            rdma.start()
            sends.append(rdma)

    for di, py in enumerate(y_peers):
        for ti, buf in enumerate(bufs):
            recv = pltpu.make_async_remote_copy(
                src_ref=buf.at[py],
                dst_ref=buf.at[py],
                send_sem=y_send_sems.at[di, ti],
                recv_sem=y_recv_sems.at[py, ti],
                device_id=(my_x, my_y, my_z),
                device_id_type=pl.DeviceIdType.MESH,
            )
            recv.wait_recv()

    m_all = m_c[...]
    mx = jnp.max(m_all, axis=0)
    w = jnp.exp(m_all - mx[None])
    l_tot = jnp.sum(l_c[...] * w, axis=0)
    w_hd = jnp.reshape(
        lax.dot_general(
            jnp.reshape(w, (N_Y * BL, H)), e_f32,
            dimension_numbers=(((1,), (0,)), ((), ())),
            preferred_element_type=jnp.float32,
        ),
        (N_Y, BL, HD),
    )
    o_tot = jnp.sum(o_c[...] * w_hd, axis=0)
    l_hd = lax.dot_general(
        l_tot, e_f32,
        dimension_numbers=(((1,), (0,)), ((), ())),
        preferred_element_type=jnp.float32,
    )
    final = o_tot / l_hd
    out_ref[my_r] = final

    for rdma in sends:
        rdma.wait_send()

    xz_sends = []
    for di, (px, pz) in enumerate(xz_peers):
        rdma = pltpu.make_async_remote_copy(
            src_ref=out_ref.at[my_r],
            dst_ref=out_ref.at[my_r],
            send_sem=xz_send_sems.at[di],
            recv_sem=xz_recv_sems.at[my_r],
            device_id=(px, my_y, pz),
            device_id_type=pl.DeviceIdType.MESH,
        )
        rdma.start()
        xz_sends.append(rdma)

    for px, pz in xz_peers:
        rr = px * N_Z + pz
        recv = pltpu.make_async_remote_copy(
            src_ref=out_ref.at[rr],
            dst_ref=out_ref.at[rr],
            send_sem=xz_send_sems.at[0],
            recv_sem=xz_recv_sems.at[rr],
            device_id=(my_x, my_y, my_z),
            device_id_type=pl.DeviceIdType.MESH,
        )
        recv.wait_recv()

    for rdma in xz_sends:
        rdma.wait_send()


def kernel(Q, K, V):
    b, skv, h, d = K.shape
    hd = h * d
    Q2 = jnp.reshape(Q, (b, 1, hd))
    K2 = jnp.reshape(K, (b, skv, hd))
    V2 = jnp.reshape(V, (b, skv, hd))
    out = pl.pallas_call(
        _body,
        in_specs=[
            pl.BlockSpec(memory_space=pltpu.VMEM),
            pl.BlockSpec(memory_space=pl.ANY),
            pl.BlockSpec(memory_space=pl.ANY),
        ],
        out_specs=pl.BlockSpec(memory_space=pltpu.VMEM),
        out_shape=jax.ShapeDtypeStruct((N_XZ, BL, hd), jnp.float32),
        scratch_shapes=[
            pltpu.VMEM((BL, skv, hd), jnp.float32),
            pltpu.VMEM((BL, skv, hd), jnp.float32),
            pltpu.VMEM((N_Y, BL, H), jnp.float32),
            pltpu.VMEM((N_Y, BL, H), jnp.float32),
            pltpu.VMEM((N_Y, BL, hd), jnp.float32),
            pltpu.SemaphoreType.DMA((BL,)),
            pltpu.SemaphoreType.DMA((BL,)),
            pltpu.SemaphoreType.DMA((N_Y - 1, 3)),
            pltpu.SemaphoreType.DMA((N_Y, 3)),
            pltpu.SemaphoreType.DMA((N_XZ - 1,)),
            pltpu.SemaphoreType.DMA((N_XZ,)),
        ],
        compiler_params=pltpu.CompilerParams(
            collective_id=0,
            vmem_limit_bytes=110 * 1024 * 1024,
        ),
    )(Q2, K2, V2)
    return jnp.reshape(out, (b, 1, h, d))
